# baseline (speedup 1.0000x reference)
import jax
import jax.numpy as jnp
from jax.experimental import pallas as pl
from jax.experimental.pallas import tpu as pltpu

B = 512
DEPTH = 3


def _body(x_hbm, pe_hbm, o_hbm, pbuf, xb0, xb1, xb2, ob0, ob1, ob2,
          sem_pe, si0, si1, si2, so0, so1, so2):
    xbufs = (xb0, xb1, xb2)
    obufs = (ob0, ob1, ob2)
    sis = (si0, si1, si2)
    sos = (so0, so1, so2)
    n_blocks = x_hbm.shape[1] // B

    def in_copy(g):
        s = g % DEPTH
        return pltpu.make_async_copy(x_hbm.at[:, pl.ds(g * B, B), :], xbufs[s], sis[s])

    def out_copy(g):
        s = g % DEPTH
        return pltpu.make_async_copy(obufs[s], o_hbm.at[:, pl.ds(g * B, B), :], sos[s])

    pe_cp = pltpu.make_async_copy(pe_hbm, pbuf, sem_pe)
    pe_cp.start()
    for k in range(DEPTH):
        in_copy(k).start()
    pe_cp.wait()
    for g in range(n_blocks):
        s = g % DEPTH
        in_copy(g).wait()
        if g >= DEPTH:
            out_copy(g - DEPTH).wait()
        obufs[s][...] = xbufs[s][...] + pbuf[pl.ds(g * B, B), :][None, :, :]
        out_copy(g).start()
        if g + DEPTH < n_blocks:
            in_copy(g + DEPTH).start()
    for g in range(n_blocks - DEPTH, n_blocks):
        out_copy(g).wait()


def kernel(x, pe):
    batch, seq_len, d_model = x.shape
    return pl.pallas_call(
        _body,
        in_specs=[
            pl.BlockSpec(memory_space=pltpu.HBM),
            pl.BlockSpec(memory_space=pltpu.HBM),
        ],
        out_specs=pl.BlockSpec(memory_space=pltpu.HBM),
        out_shape=jax.ShapeDtypeStruct((batch, seq_len, d_model), x.dtype),
        scratch_shapes=[
            pltpu.VMEM((seq_len, d_model), jnp.float32),
            pltpu.VMEM((batch, B, d_model), jnp.float32),
            pltpu.VMEM((batch, B, d_model), jnp.float32),
            pltpu.VMEM((batch, B, d_model), jnp.float32),
            pltpu.VMEM((batch, B, d_model), jnp.float32),
            pltpu.VMEM((batch, B, d_model), jnp.float32),
            pltpu.VMEM((batch, B, d_model), jnp.float32),
            pltpu.SemaphoreType.DMA,
            pltpu.SemaphoreType.DMA,
            pltpu.SemaphoreType.DMA,
            pltpu.SemaphoreType.DMA,
            pltpu.SemaphoreType.DMA,
            pltpu.SemaphoreType.DMA,
            pltpu.SemaphoreType.DMA,
        ],
        compiler_params=pltpu.CompilerParams(
            vmem_limit_bytes=110 * 1024 * 1024,
        ),
    )(x, pe[:seq_len])


# speedup vs baseline: 1.0037x; 1.0037x over previous
import jax
import jax.numpy as jnp
from jax.experimental import pallas as pl
from jax.experimental.pallas import tpu as pltpu

B = 256
DEPTH = 6


def _body(x_hbm, pe_hbm, o_hbm, *scr):
    pbuf = scr[0]
    xbufs = scr[1:1 + DEPTH]
    obufs = scr[1 + DEPTH:1 + 2 * DEPTH]
    sem_pe = scr[1 + 2 * DEPTH]
    sis = scr[2 + 2 * DEPTH:2 + 3 * DEPTH]
    sos = scr[2 + 3 * DEPTH:2 + 4 * DEPTH]
    n_blocks = x_hbm.shape[1] // B

    def in_copy(g):
        s = g % DEPTH
        return pltpu.make_async_copy(x_hbm.at[:, pl.ds(g * B, B), :], xbufs[s], sis[s])

    def out_copy(g):
        s = g % DEPTH
        return pltpu.make_async_copy(obufs[s], o_hbm.at[:, pl.ds(g * B, B), :], sos[s])

    pe_cp = pltpu.make_async_copy(pe_hbm, pbuf, sem_pe)
    pe_cp.start()
    for k in range(DEPTH):
        in_copy(k).start()
    pe_cp.wait()
    for g in range(n_blocks):
        s = g % DEPTH
        in_copy(g).wait()
        if g >= DEPTH:
            out_copy(g - DEPTH).wait()
        obufs[s][...] = xbufs[s][...] + pbuf[pl.ds(g * B, B), :][None, :, :]
        out_copy(g).start()
        if g + DEPTH < n_blocks:
            in_copy(g + DEPTH).start()
    for g in range(n_blocks - DEPTH, n_blocks):
        out_copy(g).wait()


def kernel(x, pe):
    batch, seq_len, d_model = x.shape
    scratch = (
        [pltpu.VMEM((seq_len, d_model), jnp.float32)]
        + [pltpu.VMEM((batch, B, d_model), jnp.float32) for _ in range(2 * DEPTH)]
        + [pltpu.SemaphoreType.DMA for _ in range(1 + 2 * DEPTH)]
    )
    return pl.pallas_call(
        _body,
        in_specs=[
            pl.BlockSpec(memory_space=pltpu.HBM),
            pl.BlockSpec(memory_space=pltpu.HBM),
        ],
        out_specs=pl.BlockSpec(memory_space=pltpu.HBM),
        out_shape=jax.ShapeDtypeStruct((batch, seq_len, d_model), x.dtype),
        scratch_shapes=scratch,
        compiler_params=pltpu.CompilerParams(
            vmem_limit_bytes=64 * 1024 * 1024,
        ),
    )(x, pe[:seq_len])


# speedup vs baseline: 1.0062x; 1.0025x over previous
import jax
import jax.numpy as jnp
from jax.experimental import pallas as pl
from jax.experimental.pallas import tpu as pltpu

B = 256
DEPTH = 6


def _body(x_hbm, pe_hbm, o_hbm, *scr):
    pbuf = scr[0]
    xbufs = scr[1:1 + DEPTH]
    obufs = scr[1 + DEPTH:1 + 2 * DEPTH]
    sem_pe = scr[1 + 2 * DEPTH]
    sis = scr[2 + 2 * DEPTH:2 + 3 * DEPTH]
    sos = scr[2 + 3 * DEPTH:2 + 4 * DEPTH]
    n_blocks = x_hbm.shape[1] // B

    nb = x_hbm.shape[0]

    class _Group:
        def __init__(self, copies):
            self.copies = copies

        def start(self):
            for c in self.copies:
                c.start()

        def wait(self):
            for c in self.copies:
                c.wait()

    def in_copy(g):
        s = g % DEPTH
        return _Group([
            pltpu.make_async_copy(
                x_hbm.at[b, pl.ds(g * B, B), :], xbufs[s].at[b], sis[s])
            for b in range(nb)
        ])

    def out_copy(g):
        s = g % DEPTH
        return _Group([
            pltpu.make_async_copy(
                obufs[s].at[b], o_hbm.at[b, pl.ds(g * B, B), :], sos[s])
            for b in range(nb)
        ])

    pe_cp = pltpu.make_async_copy(pe_hbm, pbuf, sem_pe)
    pe_cp.start()
    for k in range(DEPTH):
        in_copy(k).start()
    pe_cp.wait()
    for g in range(n_blocks):
        s = g % DEPTH
        in_copy(g).wait()
        if g >= DEPTH:
            out_copy(g - DEPTH).wait()
        obufs[s][...] = xbufs[s][...] + pbuf[pl.ds(g * B, B), :][None, :, :]
        out_copy(g).start()
        if g + DEPTH < n_blocks:
            in_copy(g + DEPTH).start()
    for g in range(n_blocks - DEPTH, n_blocks):
        out_copy(g).wait()


def kernel(x, pe):
    batch, seq_len, d_model = x.shape
    scratch = (
        [pltpu.VMEM((seq_len, d_model), jnp.float32)]
        + [pltpu.VMEM((batch, B, d_model), jnp.float32) for _ in range(2 * DEPTH)]
        + [pltpu.SemaphoreType.DMA for _ in range(1 + 2 * DEPTH)]
    )
    return pl.pallas_call(
        _body,
        in_specs=[
            pl.BlockSpec(memory_space=pltpu.HBM),
            pl.BlockSpec(memory_space=pltpu.HBM),
        ],
        out_specs=pl.BlockSpec(memory_space=pltpu.HBM),
        out_shape=jax.ShapeDtypeStruct((batch, seq_len, d_model), x.dtype),
        scratch_shapes=scratch,
        compiler_params=pltpu.CompilerParams(
            vmem_limit_bytes=64 * 1024 * 1024,
        ),
    )(x, pe[:seq_len])
